# baseline (speedup 1.0000x reference)
import functools

import jax
import jax.numpy as jnp
from jax import lax
from jax.experimental import pallas as pl
from jax.experimental.pallas import tpu as pltpu
from jax.experimental.pallas import tpu_sc as plsc

_B = 16384
_D = 16
_NW = 32
_BPW = _B // _NW
_CHUNK = 128
_NCH = _BPW // _CHUNK
_WD = 1e-05

_mesh = plsc.VectorSubcoreMesh(core_axis_name="c", subcore_axis_name="s")
_params = pltpu.CompilerParams(needs_layout_passes=False,
                               use_tc_tiling_on_sc=False)


def _vsqrt(x):
    i = lax.bitcast_convert_type(x, jnp.int32)
    y = lax.bitcast_convert_type(jnp.int32(0x5F3759DF) - (i >> 1), jnp.float32)
    for _ in range(3):
        y = y * (1.5 - 0.5 * x * y * y)
    return x * y


@functools.partial(
    pl.kernel,
    out_type=[
        jax.ShapeDtypeStruct((_B, _D), jnp.float32),
        jax.ShapeDtypeStruct((_B,), jnp.float32),
    ],
    mesh=_mesh,
    compiler_params=_params,
    scratch_types=[
        pltpu.VMEM((_NCH, _CHUNK), jnp.int32),
        pltpu.VMEM((_BPW, _D), jnp.float32),
        pltpu.VMEM((_BPW,), jnp.float32),
        pltpu.SemaphoreType.DMA,
    ],
)
def _sc_gather_norm(idx_hbm, tbl_hbm, rows_hbm, ss_hbm,
                    idx_v, rows, ss_v, sem):
    cid = lax.axis_index("c")
    sid = lax.axis_index("s")
    wid = sid * 2 + cid
    base = wid * _BPW

    for ch in range(_NCH):
        pltpu.sync_copy(idx_hbm.at[pl.ds(base + ch * _CHUNK, _CHUNK)],
                        idx_v.at[ch])
    copies = []
    for ch in range(_NCH):
        copies.append(pltpu.async_copy(
            tbl_hbm.at[idx_v.at[ch]],
            rows.at[pl.ds(ch * _CHUNK, _CHUNK)], sem))
    for cp in copies:
        cp.wait()

    iota = lax.iota(jnp.int32, 16)

    def block(r, carry):
        rb = r * 16
        row_idx = rb + iota
        acc = jnp.zeros((16,), jnp.float32)
        for j in range(_D):
            cj = jnp.full((16,), j, jnp.int32)
            g = plsc.load_gather(rows, [row_idx, cj])
            acc = acc + g * g
        ss_v[pl.ds(rb, 16)] = acc
        return carry

    lax.fori_loop(0, _BPW // 16, block, 0)

    pltpu.sync_copy(rows, rows_hbm.at[pl.ds(base, _BPW)])
    pltpu.sync_copy(ss_v, ss_hbm.at[pl.ds(base, _BPW)])


@functools.partial(
    pl.kernel,
    out_type=[
        jax.ShapeDtypeStruct((_B, _D), jnp.float32),
        jax.ShapeDtypeStruct((_B,), jnp.float32),
    ],
    mesh=_mesh,
    compiler_params=_params,
    scratch_types=[
        pltpu.VMEM((_BPW, _D), jnp.float32),
        pltpu.VMEM((_BPW, _D), jnp.float32),
        pltpu.VMEM((_BPW, _D), jnp.float32),
        pltpu.VMEM((_BPW,), jnp.float32),
        pltpu.VMEM((_BPW,), jnp.float32),
        pltpu.VMEM((_BPW,), jnp.float32),
    ],
)
def _sc_combine(xr_hbm, ar_hbm, ssx_hbm, ssa_hbm, prod_hbm, wd_hbm,
                xr, ar, prod, ssx, ssa, wd_v):
    cid = lax.axis_index("c")
    sid = lax.axis_index("s")
    wid = sid * 2 + cid
    base = wid * _BPW

    pltpu.sync_copy(xr_hbm.at[pl.ds(base, _BPW)], xr)
    pltpu.sync_copy(ar_hbm.at[pl.ds(base, _BPW)], ar)
    pltpu.sync_copy(ssx_hbm.at[pl.ds(base, _BPW)], ssx)
    pltpu.sync_copy(ssa_hbm.at[pl.ds(base, _BPW)], ssa)

    def pblock(r, carry):
        rb = r * 16
        for k in range(16):
            i = rb + k
            prod[i, :] = xr[i, :] * ar[i, :]
        wd_v[pl.ds(rb, 16)] = _WD * (_vsqrt(ssx[pl.ds(rb, 16)]) +
                                     _vsqrt(ssa[pl.ds(rb, 16)]))
        return carry

    lax.fori_loop(0, _BPW // 16, pblock, 0)

    pltpu.sync_copy(prod, prod_hbm.at[pl.ds(base, _BPW)])
    pltpu.sync_copy(wd_v, wd_hbm.at[pl.ds(base, _BPW)])


def kernel(x_raw, x_table, a_table):
    idx = x_raw.astype(jnp.int32)
    xr, ssx = _sc_gather_norm(idx[:, 0], x_table)
    ar, ssa = _sc_gather_norm(idx[:, 1], a_table)
    prod, wd = _sc_combine(xr, ar, ssx, ssa)
    return (prod, wd)


# speedup vs baseline: 1.2952x; 1.2952x over previous
import functools

import jax
import jax.numpy as jnp
from jax import lax
from jax.experimental import pallas as pl
from jax.experimental.pallas import tpu as pltpu
from jax.experimental.pallas import tpu_sc as plsc

_B = 16384
_D = 16
_NW = 32
_BPW = _B // _NW
_WD = 1e-05

_mesh = plsc.VectorSubcoreMesh(core_axis_name="c", subcore_axis_name="s")


def _vsqrt(x):
    i = lax.bitcast_convert_type(x, jnp.int32)
    y = lax.bitcast_convert_type(jnp.int32(0x5F3759DF) - (i >> 1), jnp.float32)
    for _ in range(3):
        y = y * (1.5 - 0.5 * x * y * y)
    return x * y


@functools.partial(
    pl.kernel,
    out_type=[
        jax.ShapeDtypeStruct((_B // 8, 128), jnp.float32),
        jax.ShapeDtypeStruct((_B,), jnp.float32),
    ],
    mesh=_mesh,
    compiler_params=pltpu.CompilerParams(needs_layout_passes=False,
                                         use_tc_tiling_on_sc=True),
    scratch_types=[
        pltpu.VMEM((4, 128), jnp.int32),
        pltpu.VMEM((4, 128), jnp.int32),
        pltpu.VMEM((16, 8, 16), jnp.float32),
        pltpu.VMEM((16, 8, 16), jnp.float32),
        pltpu.VMEM((64, 128), jnp.float32),
        pltpu.VMEM((64, 128), jnp.float32),
        pltpu.VMEM((64, 128), jnp.float32),
        pltpu.VMEM((_BPW,), jnp.float32),
        pltpu.SemaphoreType.DMA,
    ],
)
def _sc_embed(idxx_hbm, idxa_hbm, xt_hbm, at_hbm, op_hbm, owd_hbm,
              idxx_v, idxa_v, ringx, ringa, xrows, arows, prod, wd_v, sem):
    cid = lax.axis_index("c")
    sid = lax.axis_index("s")
    wid = sid * 2 + cid
    base = wid * _BPW

    for ch in range(4):
        pltpu.sync_copy(idxx_hbm.at[pl.ds(base + ch * 128, 128)],
                        idxx_v.at[ch])
        pltpu.sync_copy(idxa_hbm.at[pl.ds(base + ch * 128, 128)],
                        idxa_v.at[ch])

    def make_fetch(idx_v, tbl_hbm, ring, rows):
        def fetch_chunk(q, carry):
            gvec = idx_v[q // 8, pl.ds((q % 8) * 16, 16)]
            cps = []
            for k in range(16):
                g8 = (gvec[k] >> 3) * 8
                cps.append(pltpu.async_copy(
                    tbl_hbm.at[pl.ds(g8, 8)], ring.at[k], sem))
            for k in range(16):
                cps[k].wait()
                j = gvec[k] & 7
                rows[q * 2 + k // 8, pl.ds((k % 8) * 16, 16)] = ring[k, j, :]
            return carry
        return fetch_chunk

    lax.fori_loop(0, _BPW // 16, make_fetch(idxx_v, xt_hbm, ringx, xrows), 0)
    lax.fori_loop(0, _BPW // 16, make_fetch(idxa_v, at_hbm, ringa, arows), 0)

    iota = lax.iota(jnp.int32, 16)

    def block(b, carry):
        rb = b * 16
        rvec = rb + iota
        r2 = rvec >> 3
        c0 = (rvec & 7) * 16
        accx = jnp.zeros((16,), jnp.float32)
        acca = jnp.zeros((16,), jnp.float32)
        for j in range(_D):
            cj = c0 + j
            gx = plsc.load_gather(xrows, [r2, cj])
            ga = plsc.load_gather(arows, [r2, cj])
            plsc.store_scatter(prod, [r2, cj], gx * ga)
            accx = accx + gx * gx
            acca = acca + ga * ga
        wd_v[pl.ds(rb, 16)] = _WD * (_vsqrt(accx) + _vsqrt(acca))
        return carry

    lax.fori_loop(0, _BPW // 16, block, 0)

    pltpu.sync_copy(prod, op_hbm.at[pl.ds(wid * 64, 64)])
    pltpu.sync_copy(wd_v, owd_hbm.at[pl.ds(base, _BPW)])


def kernel(x_raw, x_table, a_table):
    idx = x_raw.astype(jnp.int32)
    packed, wd = _sc_embed(idx[:, 0], idx[:, 1], x_table, a_table)
    return (packed.reshape(_B, _D), wd)


# speedup vs baseline: 4.5409x; 3.5059x over previous
import functools

import jax
import jax.numpy as jnp
from jax import lax
from jax.experimental import pallas as pl
from jax.experimental.pallas import tpu as pltpu
from jax.experimental.pallas import tpu_sc as plsc

_B = 16384
_D = 16
_NW = 32
_BPW = _B // _NW
_V = 1000000
_WLAST = (_V // 128 - 1) * 128
_TAIL0 = (_V // 128) * 128
_NTAIL = _V - _TAIL0
_WD = 1e-05

_mesh = plsc.VectorSubcoreMesh(core_axis_name="c", subcore_axis_name="s")


def _vsqrt(x):
    i = lax.bitcast_convert_type(x, jnp.int32)
    y = lax.bitcast_convert_type(jnp.int32(0x5F3759DF) - (i >> 1), jnp.float32)
    for _ in range(3):
        y = y * (1.5 - 0.5 * x * y * y)
    return x * y


@functools.partial(
    pl.kernel,
    out_type=[
        jax.ShapeDtypeStruct((_B // 8, 128), jnp.float32),
        jax.ShapeDtypeStruct((_B,), jnp.float32),
    ],
    mesh=_mesh,
    compiler_params=pltpu.CompilerParams(needs_layout_passes=False,
                                         use_tc_tiling_on_sc=True),
    scratch_types=[
        pltpu.VMEM((4, 128), jnp.int32),
        pltpu.VMEM((4, 128), jnp.int32),
        pltpu.VMEM((16, 16, 128), jnp.float32),
        pltpu.VMEM((16, 16, 128), jnp.float32),
        pltpu.VMEM((16, _NTAIL), jnp.float32),
        pltpu.VMEM((16, _NTAIL), jnp.float32),
        pltpu.VMEM((64, 128), jnp.float32),
        pltpu.VMEM((64, 128), jnp.float32),
        pltpu.VMEM((64, 128), jnp.float32),
        pltpu.VMEM((_BPW,), jnp.float32),
        pltpu.SemaphoreType.DMA,
    ],
)
def _sc_embed(idxx_hbm, idxa_hbm, xt_hbm, at_hbm, op_hbm, owd_hbm,
              idxx_v, idxa_v, ringx, ringa, tailx, taila,
              xrows, arows, prod, wd_v, sem):
    cid = lax.axis_index("c")
    sid = lax.axis_index("s")
    wid = sid * 2 + cid
    base = wid * _BPW

    for ch in range(4):
        pltpu.sync_copy(idxx_hbm.at[pl.ds(base + ch * 128, 128)],
                        idxx_v.at[ch])
        pltpu.sync_copy(idxa_hbm.at[pl.ds(base + ch * 128, 128)],
                        idxa_v.at[ch])
    pltpu.sync_copy(xt_hbm.at[:, pl.ds(_TAIL0, _NTAIL)], tailx)
    pltpu.sync_copy(at_hbm.at[:, pl.ds(_TAIL0, _NTAIL)], taila)

    iota = lax.iota(jnp.int32, 16)

    def make_fetch(idx_v, tt_hbm, ring, tail, rows):
        def fetch_chunk(q, carry):
            gvec = idx_v[q // 8, pl.ds((q % 8) * 16, 16)]
            cps = []
            for k in range(16):
                w0 = jnp.minimum((gvec[k] >> 7) * 128, _WLAST)
                cps.append(pltpu.async_copy(
                    tt_hbm.at[:, pl.ds(w0, 128)], ring.at[k], sem))
            for k in range(16):
                cps[k].wait()
                g = gvec[k]
                c = g - jnp.minimum((g >> 7) * 128, _WLAST)
                kf = jnp.full((16,), k, jnp.int32)
                cr = jnp.full((16,), jnp.minimum(c, 127), jnp.int32)
                ct = jnp.full((16,), jnp.maximum(c - 128, 0), jnp.int32)
                rv = plsc.load_gather(ring, [kf, iota, cr])
                tv = plsc.load_gather(tail, [iota, ct])
                rows[q * 2 + k // 8, pl.ds((k % 8) * 16, 16)] = (
                    jnp.where(c < 128, rv, tv))
            return carry
        return fetch_chunk

    lax.fori_loop(0, _BPW // 16,
                  make_fetch(idxx_v, xt_hbm, ringx, tailx, xrows), 0)
    lax.fori_loop(0, _BPW // 16,
                  make_fetch(idxa_v, at_hbm, ringa, taila, arows), 0)

    def block(b, carry):
        rb = b * 16
        rvec = rb + iota
        r2 = rvec >> 3
        c0 = (rvec & 7) * 16
        accx = jnp.zeros((16,), jnp.float32)
        acca = jnp.zeros((16,), jnp.float32)
        for j in range(_D):
            cj = c0 + j
            gx = plsc.load_gather(xrows, [r2, cj])
            ga = plsc.load_gather(arows, [r2, cj])
            plsc.store_scatter(prod, [r2, cj], gx * ga)
            accx = accx + gx * gx
            acca = acca + ga * ga
        wd_v[pl.ds(rb, 16)] = _WD * (_vsqrt(accx) + _vsqrt(acca))
        return carry

    lax.fori_loop(0, _BPW // 16, block, 0)

    pltpu.sync_copy(prod, op_hbm.at[pl.ds(wid * 64, 64)])
    pltpu.sync_copy(wd_v, owd_hbm.at[pl.ds(base, _BPW)])


def kernel(x_raw, x_table, a_table):
    idx = x_raw.astype(jnp.int32)
    packed, wd = _sc_embed(idx[:, 0], idx[:, 1], x_table.T, a_table.T)
    return (packed.reshape(_B, _D), wd)


# speedup vs baseline: 5.3329x; 1.1744x over previous
import functools

import jax
import jax.numpy as jnp
from jax import lax
from jax.experimental import pallas as pl
from jax.experimental.pallas import tpu as pltpu
from jax.experimental.pallas import tpu_sc as plsc

_B = 16384
_D = 16
_NW = 32
_BPW = _B // _NW
_V = 1000000
_WLAST = (_V // 128 - 1) * 128
_TAIL0 = (_V // 128) * 128
_NTAIL = _V - _TAIL0
_WD = 1e-05

_mesh = plsc.VectorSubcoreMesh(core_axis_name="c", subcore_axis_name="s")


def _vsqrt(x):
    i = lax.bitcast_convert_type(x, jnp.int32)
    y = lax.bitcast_convert_type(jnp.int32(0x5F3759DF) - (i >> 1), jnp.float32)
    for _ in range(3):
        y = y * (1.5 - 0.5 * x * y * y)
    return x * y


@functools.partial(
    pl.kernel,
    out_type=[
        jax.ShapeDtypeStruct((_B // 8, 128), jnp.float32),
        jax.ShapeDtypeStruct((_B,), jnp.float32),
    ],
    mesh=_mesh,
    compiler_params=pltpu.CompilerParams(needs_layout_passes=False,
                                         use_tc_tiling_on_sc=True),
    scratch_types=[
        pltpu.VMEM((4, 128), jnp.int32),
        pltpu.VMEM((4, 128), jnp.int32),
        pltpu.VMEM((16, 16, 128), jnp.float32),
        pltpu.VMEM((16, 16, 128), jnp.float32),
        pltpu.VMEM((16, _NTAIL), jnp.float32),
        pltpu.VMEM((16, _NTAIL), jnp.float32),
        pltpu.VMEM((64, 128), jnp.float32),
        pltpu.VMEM((64, 128), jnp.float32),
        pltpu.VMEM((64, 128), jnp.float32),
        pltpu.VMEM((_BPW,), jnp.float32),
        pltpu.SemaphoreType.DMA,
    ],
)
def _sc_embed(idxx_hbm, idxa_hbm, xt_hbm, at_hbm, op_hbm, owd_hbm,
              idxx_v, idxa_v, ringx, ringa, tailx, taila,
              xrows, arows, prod, wd_v, sem):
    cid = lax.axis_index("c")
    sid = lax.axis_index("s")
    wid = sid * 2 + cid
    base = wid * _BPW

    for ch in range(4):
        pltpu.sync_copy(idxx_hbm.at[pl.ds(base + ch * 128, 128)],
                        idxx_v.at[ch])
        pltpu.sync_copy(idxa_hbm.at[pl.ds(base + ch * 128, 128)],
                        idxa_v.at[ch])
    pltpu.sync_copy(xt_hbm.at[:, pl.ds(_TAIL0, _NTAIL)], tailx)
    pltpu.sync_copy(at_hbm.at[:, pl.ds(_TAIL0, _NTAIL)], taila)

    iota = lax.iota(jnp.int32, 16)

    def issue16(gvec, tt_hbm, ring):
        cps = []
        for k in range(16):
            w0 = jnp.minimum((gvec[k] >> 7) * 128, _WLAST)
            cps.append(pltpu.async_copy(
                tt_hbm.at[:, pl.ds(w0, 128)], ring.at[k], sem))
        return cps

    def drain16(q, gvec, cps, ring, tail, rows):
        for k in range(16):
            cps[k].wait()
            g = gvec[k]
            c = g - jnp.minimum((g >> 7) * 128, _WLAST)
            kf = jnp.full((16,), k, jnp.int32)
            cr = jnp.full((16,), jnp.minimum(c, 127), jnp.int32)
            ct = jnp.full((16,), jnp.maximum(c - 128, 0), jnp.int32)
            rv = plsc.load_gather(ring, [kf, iota, cr])
            tv = plsc.load_gather(tail, [iota, ct])
            rows[q * 2 + k // 8, pl.ds((k % 8) * 16, 16)] = (
                jnp.where(c < 128, rv, tv))

    def fetch_chunk(q, carry):
        gxv = idxx_v[q // 8, pl.ds((q % 8) * 16, 16)]
        gav = idxa_v[q // 8, pl.ds((q % 8) * 16, 16)]
        cpx = issue16(gxv, xt_hbm, ringx)
        cpa = issue16(gav, at_hbm, ringa)
        drain16(q, gxv, cpx, ringx, tailx, xrows)
        drain16(q, gav, cpa, ringa, taila, arows)
        return carry

    lax.fori_loop(0, _BPW // 16, fetch_chunk, 0)

    def block(b, carry):
        rb = b * 16
        rvec = rb + iota
        r2 = rvec >> 3
        c0 = (rvec & 7) * 16
        accx = jnp.zeros((16,), jnp.float32)
        acca = jnp.zeros((16,), jnp.float32)
        for j in range(_D):
            cj = c0 + j
            gx = plsc.load_gather(xrows, [r2, cj])
            ga = plsc.load_gather(arows, [r2, cj])
            plsc.store_scatter(prod, [r2, cj], gx * ga)
            accx = accx + gx * gx
            acca = acca + ga * ga
        wd_v[pl.ds(rb, 16)] = _WD * (_vsqrt(accx) + _vsqrt(acca))
        return carry

    lax.fori_loop(0, _BPW // 16, block, 0)

    pltpu.sync_copy(prod, op_hbm.at[pl.ds(wid * 64, 64)])
    pltpu.sync_copy(wd_v, owd_hbm.at[pl.ds(base, _BPW)])


def kernel(x_raw, x_table, a_table):
    idx = x_raw.astype(jnp.int32)
    packed, wd = _sc_embed(idx[:, 0], idx[:, 1], x_table.T, a_table.T)
    return (packed.reshape(_B, _D), wd)


# speedup vs baseline: 5.3656x; 1.0061x over previous
import functools

import jax
import jax.numpy as jnp
from jax import lax
from jax.experimental import pallas as pl
from jax.experimental.pallas import tpu as pltpu
from jax.experimental.pallas import tpu_sc as plsc

_B = 16384
_D = 16
_NW = 32
_BPW = _B // _NW
_V = 1000000
_WLAST = (_V // 128 - 1) * 128
_TAIL0 = (_V // 128) * 128
_NTAIL = _V - _TAIL0
_WD = 1e-05

_mesh = plsc.VectorSubcoreMesh(core_axis_name="c", subcore_axis_name="s")


def _vsqrt(x):
    i = lax.bitcast_convert_type(x, jnp.int32)
    y = lax.bitcast_convert_type(jnp.int32(0x5F3759DF) - (i >> 1), jnp.float32)
    for _ in range(3):
        y = y * (1.5 - 0.5 * x * y * y)
    return x * y


@functools.partial(
    pl.kernel,
    out_type=[
        jax.ShapeDtypeStruct((_B // 8, 128), jnp.float32),
        jax.ShapeDtypeStruct((_B,), jnp.float32),
    ],
    mesh=_mesh,
    compiler_params=pltpu.CompilerParams(needs_layout_passes=False,
                                         use_tc_tiling_on_sc=True),
    scratch_types=[
        pltpu.VMEM((4, 128), jnp.int32),
        pltpu.VMEM((4, 128), jnp.int32),
        pltpu.VMEM((16, 16, 128), jnp.float32),
        pltpu.VMEM((16, 16, 128), jnp.float32),
        pltpu.VMEM((16, _NTAIL), jnp.float32),
        pltpu.VMEM((16, _NTAIL), jnp.float32),
        pltpu.VMEM((64, 128), jnp.float32),
        pltpu.VMEM((64, 128), jnp.float32),
        pltpu.VMEM((64, 128), jnp.float32),
        pltpu.VMEM((_BPW,), jnp.float32),
        pltpu.SemaphoreType.DMA,
    ],
)
def _sc_embed(idxx_hbm, idxa_hbm, xt_hbm, at_hbm, op_hbm, owd_hbm,
              idxx_v, idxa_v, ringx, ringa, tailx, taila,
              xrows, arows, prod, wd_v, sem):
    cid = lax.axis_index("c")
    sid = lax.axis_index("s")
    wid = sid * 2 + cid
    base = wid * _BPW

    for ch in range(4):
        pltpu.sync_copy(idxx_hbm.at[pl.ds(base + ch * 128, 128)],
                        idxx_v.at[ch])
        pltpu.sync_copy(idxa_hbm.at[pl.ds(base + ch * 128, 128)],
                        idxa_v.at[ch])
    pltpu.sync_copy(xt_hbm.at[:, pl.ds(_TAIL0, _NTAIL)], tailx)
    pltpu.sync_copy(at_hbm.at[:, pl.ds(_TAIL0, _NTAIL)], taila)

    iota = lax.iota(jnp.int32, 16)

    def issue16(gvec, tt_hbm, ring):
        cps = []
        for k in range(16):
            w0 = jnp.minimum((gvec[k] >> 7) * 128, _WLAST)
            cps.append(pltpu.async_copy(
                tt_hbm.at[:, pl.ds(w0, 128)], ring.at[k], sem))
        return cps

    def drain16(q, gvec, cps, ring, tail, rows):
        for k in range(16):
            cps[k].wait()
            g = gvec[k]
            c = g - jnp.minimum((g >> 7) * 128, _WLAST)
            kf = jnp.full((16,), k, jnp.int32)
            cr = jnp.full((16,), jnp.minimum(c, 127), jnp.int32)
            ct = jnp.full((16,), jnp.maximum(c - 128, 0), jnp.int32)
            rv = plsc.load_gather(ring, [kf, iota, cr])
            tv = plsc.load_gather(tail, [iota, ct])
            rows[q * 2 + k // 8, pl.ds((k % 8) * 16, 16)] = (
                jnp.where(c < 128, rv, tv))

    def fetch_chunk(q, carry):
        gxv = idxx_v[q // 8, pl.ds((q % 8) * 16, 16)]
        gav = idxa_v[q // 8, pl.ds((q % 8) * 16, 16)]
        cpx = issue16(gxv, xt_hbm, ringx)
        cpa = issue16(gav, at_hbm, ringa)
        drain16(q, gxv, cpx, ringx, tailx, xrows)
        drain16(q, gav, cpa, ringa, taila, arows)
        rb = q * 16
        rvec = rb + iota
        r2 = rvec >> 3
        c0 = (rvec & 7) * 16
        accx = jnp.zeros((16,), jnp.float32)
        acca = jnp.zeros((16,), jnp.float32)
        for j in range(_D):
            cj = c0 + j
            gx = plsc.load_gather(xrows, [r2, cj])
            ga = plsc.load_gather(arows, [r2, cj])
            plsc.store_scatter(prod, [r2, cj], gx * ga)
            accx = accx + gx * gx
            acca = acca + ga * ga
        wd_v[pl.ds(rb, 16)] = _WD * (_vsqrt(accx) + _vsqrt(acca))
        return carry

    lax.fori_loop(0, _BPW // 16, fetch_chunk, 0)

    pltpu.sync_copy(prod, op_hbm.at[pl.ds(wid * 64, 64)])
    pltpu.sync_copy(wd_v, owd_hbm.at[pl.ds(base, _BPW)])


def kernel(x_raw, x_table, a_table):
    idx = x_raw.astype(jnp.int32)
    packed, wd = _sc_embed(idx[:, 0], idx[:, 1], x_table.T, a_table.T)
    return (packed.reshape(_B, _D), wd)
